# Initial kernel scaffold; baseline (speedup 1.0000x reference)
import functools

import jax
import jax.numpy as jnp
from jax import lax
from jax.experimental import pallas as pl
from jax.experimental.pallas import tpu as pltpu
from jax.experimental.pallas import tpu_sc as plsc

N_NODES = 10000
N_EDGES = 320000
D_IN = 128
D_OUT = 128

D_AUG = 144
DEG_COL = 128

NC = 2
NS = 16
NW = NC * NS

CHUNK = 128
EDGES_PER_WORKER = 10240
NCHUNKS = EDGES_PER_WORKER // CHUNK
E_PAD = EDGES_PER_WORKER * NW

N_ACC = 10240
ROWS_PT = N_ACC // NS
DUMMY = N_NODES

BM = 2000


def _sc_aggregate(x_aug, src2d, dst2d):
  mesh = plsc.VectorSubcoreMesh(core_axis_name="c", subcore_axis_name="s")

  @functools.partial(
      pl.kernel,
      out_type=jax.ShapeDtypeStruct((NC, N_ACC, D_AUG), jnp.float32),
      mesh=mesh,
      scratch_types=[
          pltpu.VMEM((NCHUNKS, CHUNK), jnp.int32),
          pltpu.VMEM((NCHUNKS, CHUNK), jnp.int32),
          pltpu.VMEM((CHUNK, D_AUG), jnp.float32),
          pltpu.VMEM_SHARED((N_ACC, D_AUG), jnp.float32),
          pltpu.SemaphoreType.DMA,
      ],
  )
  def agg(x_ref, src_ref, dst_ref, out_ref, src_v, dst_v, rows_v, acc_sp, sem):
    c = lax.axis_index("c")
    s = lax.axis_index("s")
    w = c * NS + s
    base = s * ROWS_PT

    zeros = jnp.zeros((16,), jnp.float32)
    nseg = D_AUG // 16

    def zbody(i, carry):
      rows_v[i // nseg, pl.ds((i % nseg) * 16, 16)] = zeros
      return carry

    lax.fori_loop(0, CHUNK * nseg, zbody, 0)

    def zcopy(i, carry):
      pltpu.sync_copy(rows_v, acc_sp.at[pl.ds(base + i * CHUNK, CHUNK)])
      return carry

    lax.fori_loop(0, ROWS_PT // CHUNK, zcopy, 0)

    pltpu.sync_copy(src_ref.at[pl.ds(w * NCHUNKS, NCHUNKS)], src_v)
    pltpu.sync_copy(dst_ref.at[pl.ds(w * NCHUNKS, NCHUNKS)], dst_v)

    plsc.subcore_barrier()

    def body(j, carry):
      pltpu.async_copy(x_ref.at[src_v.at[j]], rows_v, sem).wait()
      pltpu.sync_copy(rows_v, acc_sp.at[dst_v.at[j]], add=True)
      return carry

    lax.fori_loop(0, NCHUNKS, body, 0)

    plsc.subcore_barrier()

    pltpu.sync_copy(acc_sp.at[pl.ds(base, ROWS_PT)],
                    out_ref.at[c, pl.ds(base, ROWS_PT)])

  return agg(x_aug, src2d, dst2d)


def _tc_body(x_ref, acc_ref, w_ref, m_ref, b_ref, o_ref):
  wm = w_ref[...] * m_ref[...]
  cs = acc_ref[0] + acc_ref[1]
  deg = cs[:, DEG_COL:DEG_COL + 1]
  cmean = cs[:, :D_IN] / jnp.maximum(deg, 1.0)
  h = (jnp.dot(x_ref[...], wm[:D_IN], preferred_element_type=jnp.float32)
       + jnp.dot(cmean, wm[D_IN:], preferred_element_type=jnp.float32)
       + b_ref[...])
  n = jnp.sqrt(jnp.sum(h * h, axis=1, keepdims=True))
  o_ref[...] = h / jnp.maximum(n, 1e-12)


def _tc_apply(x, acc, W, mask, b2):
  return pl.pallas_call(
      _tc_body,
      grid=(N_NODES // BM,),
      in_specs=[
          pl.BlockSpec((BM, D_IN), lambda i: (i, 0)),
          pl.BlockSpec((NC, BM, D_AUG), lambda i: (0, i, 0)),
          pl.BlockSpec((2 * D_IN, D_OUT), lambda i: (0, 0)),
          pl.BlockSpec((2 * D_IN, D_OUT), lambda i: (0, 0)),
          pl.BlockSpec((1, D_OUT), lambda i: (0, 0)),
      ],
      out_specs=pl.BlockSpec((BM, D_OUT), lambda i: (i, 0)),
      out_shape=jax.ShapeDtypeStruct((N_NODES, D_OUT), jnp.float32),
  )(x, acc, W, mask, b2)


def kernel(x, edge_index, W, b, mask):
  x = x.astype(jnp.float32)
  ei = edge_index.astype(jnp.int32)
  npad = E_PAD - N_EDGES
  src = jnp.concatenate([ei[0], jnp.zeros((npad,), jnp.int32)])
  dst = jnp.concatenate([ei[1], jnp.full((npad,), DUMMY, jnp.int32)])
  src2d = src.reshape(NW * NCHUNKS, CHUNK)
  dst2d = dst.reshape(NW * NCHUNKS, CHUNK)
  x_aug = jnp.concatenate(
      [x, jnp.ones((N_NODES, 1), jnp.float32),
       jnp.zeros((N_NODES, D_AUG - D_IN - 1), jnp.float32)], axis=1)
  acc = _sc_aggregate(x_aug, src2d, dst2d)
  return _tc_apply(x, acc, W, mask, b.reshape(1, D_OUT))


# speedup vs baseline: 4.0719x; 4.0719x over previous
import functools

import jax
import jax.numpy as jnp
from jax import lax
from jax.experimental import pallas as pl
from jax.experimental.pallas import tpu as pltpu
from jax.experimental.pallas import tpu_sc as plsc

N_NODES = 10000
N_EDGES = 320000
D_IN = 128
D_OUT = 128

D_AUG = 144
DEG_COL = 128

NC = 2
NS = 16
NW = NC * NS

CHUNK = 128
EDGES_PER_WORKER = 10240
NCHUNKS = EDGES_PER_WORKER // CHUNK
E_PAD = EDGES_PER_WORKER * NW

N_ACC = 10240
ROWS_PT = N_ACC // NS
DUMMY = N_NODES

BM = 2000


def _sc_aggregate(x_aug, src2d, dst2d):
  mesh = plsc.VectorSubcoreMesh(core_axis_name="c", subcore_axis_name="s")

  @functools.partial(
      pl.kernel,
      out_type=jax.ShapeDtypeStruct((NC, N_ACC, D_AUG), jnp.float32),
      mesh=mesh,
      compiler_params=pltpu.CompilerParams(use_tc_tiling_on_sc=False),
      scratch_types=[
          pltpu.VMEM((NCHUNKS, CHUNK), jnp.int32),
          pltpu.VMEM((NCHUNKS, CHUNK), jnp.int32),
          pltpu.VMEM((CHUNK, D_AUG), jnp.float32),
          pltpu.VMEM_SHARED((N_ACC, D_AUG), jnp.float32),
          pltpu.SemaphoreType.DMA,
      ],
  )
  def agg(x_ref, src_ref, dst_ref, out_ref, src_v, dst_v, rows_v, acc_sp, sem):
    c = lax.axis_index("c")
    s = lax.axis_index("s")
    w = c * NS + s
    base = s * ROWS_PT

    zeros = jnp.zeros((16,), jnp.float32)
    nseg = D_AUG // 16

    def zbody(i, carry):
      rows_v[i // nseg, pl.ds((i % nseg) * 16, 16)] = zeros
      return carry

    lax.fori_loop(0, CHUNK * nseg, zbody, 0)

    def zcopy(i, carry):
      pltpu.sync_copy(rows_v, acc_sp.at[pl.ds(base + i * CHUNK, CHUNK)])
      return carry

    lax.fori_loop(0, ROWS_PT // CHUNK, zcopy, 0)

    pltpu.sync_copy(src_ref.at[pl.ds(w * NCHUNKS, NCHUNKS)], src_v)
    pltpu.sync_copy(dst_ref.at[pl.ds(w * NCHUNKS, NCHUNKS)], dst_v)

    plsc.subcore_barrier()

    def body(j, carry):
      pltpu.async_copy(x_ref.at[src_v.at[j]], rows_v, sem).wait()
      pltpu.sync_copy(rows_v, acc_sp.at[dst_v.at[j]], add=True)
      return carry

    lax.fori_loop(0, NCHUNKS, body, 0)

    plsc.subcore_barrier()

    pltpu.sync_copy(acc_sp.at[pl.ds(base, ROWS_PT)],
                    out_ref.at[c, pl.ds(base, ROWS_PT)])

  return agg(x_aug, src2d, dst2d)


def _tc_body(x_ref, acc_ref, w_ref, m_ref, b_ref, o_ref):
  wm = w_ref[...] * m_ref[...]
  cs = acc_ref[0] + acc_ref[1]
  deg = cs[:, DEG_COL:DEG_COL + 1]
  cmean = cs[:, :D_IN] / jnp.maximum(deg, 1.0)
  h = (jnp.dot(x_ref[...], wm[:D_IN], preferred_element_type=jnp.float32)
       + jnp.dot(cmean, wm[D_IN:], preferred_element_type=jnp.float32)
       + b_ref[...])
  n = jnp.sqrt(jnp.sum(h * h, axis=1, keepdims=True))
  o_ref[...] = h / jnp.maximum(n, 1e-12)


def _tc_apply(x, acc, W, mask, b2):
  return pl.pallas_call(
      _tc_body,
      grid=(N_NODES // BM,),
      in_specs=[
          pl.BlockSpec((BM, D_IN), lambda i: (i, 0)),
          pl.BlockSpec((NC, BM, D_AUG), lambda i: (0, i, 0)),
          pl.BlockSpec((2 * D_IN, D_OUT), lambda i: (0, 0)),
          pl.BlockSpec((2 * D_IN, D_OUT), lambda i: (0, 0)),
          pl.BlockSpec((1, D_OUT), lambda i: (0, 0)),
      ],
      out_specs=pl.BlockSpec((BM, D_OUT), lambda i: (i, 0)),
      out_shape=jax.ShapeDtypeStruct((N_NODES, D_OUT), jnp.float32),
  )(x, acc, W, mask, b2)


def kernel(x, edge_index, W, b, mask):
  x = x.astype(jnp.float32)
  ei = edge_index.astype(jnp.int32)
  npad = E_PAD - N_EDGES
  src = jnp.concatenate([ei[0], jnp.zeros((npad,), jnp.int32)])
  dst = jnp.concatenate([ei[1], jnp.full((npad,), DUMMY, jnp.int32)])
  src2d = src.reshape(NW * NCHUNKS, CHUNK)
  dst2d = dst.reshape(NW * NCHUNKS, CHUNK)
  x_aug = jnp.concatenate(
      [x, jnp.ones((N_NODES, 1), jnp.float32),
       jnp.zeros((N_NODES, D_AUG - D_IN - 1), jnp.float32)], axis=1)
  acc = _sc_aggregate(x_aug, src2d, dst2d)
  return _tc_apply(x, acc, W, mask, b.reshape(1, D_OUT))
